# baseline (speedup 1.0000x reference)
import jax
import jax.numpy as jnp
from jax import lax
from jax.experimental import pallas as pl
from jax.experimental.pallas import tpu as pltpu
from jax.experimental.pallas import tpu_sc as plsc

_NC = 2
_NS = 16
_LANES = 16


def kernel(x, emb_weight, cls_token):
    b, L = x.shape
    D = emb_weight.shape[1]
    LD = (L + 1) * D
    nw = _NC * _NS
    rows_per_w = b // nw
    Lp = -(-(L + 1) // 8) * 8
    xa = jnp.concatenate(
        [x, jnp.ones((b, 1), jnp.float32), jnp.zeros((b, Lp - L - 1), jnp.float32)],
        axis=1,
    )
    wflat = jnp.concatenate(
        [
            emb_weight[1 : L + 1].reshape(L * D),
            cls_token.reshape(D),
            jnp.zeros((Lp - L - 1) * D, jnp.float32),
        ],
        axis=0,
    )
    LDp = Lp * D
    n_chunks = Lp // _LANES

    mesh = plsc.VectorSubcoreMesh(core_axis_name="c", subcore_axis_name="s")

    @pl.kernel(
        mesh=mesh,
        out_type=jax.ShapeDtypeStruct((b * LD,), jnp.float32),
        scratch_types=[
            pltpu.VMEM((rows_per_w * Lp,), jnp.float32),
            pltpu.VMEM((LDp,), jnp.float32),
            pltpu.VMEM((LDp,), jnp.float32),
        ],
    )
    def sc_emb(xa_hbm, w_hbm, out_hbm, x_v, w_v, row_v):
        wid = lax.axis_index("s") * _NC + lax.axis_index("c")
        base = wid * rows_per_w
        pltpu.sync_copy(xa_hbm.at[pl.ds(base * Lp, rows_per_w * Lp)], x_v)
        pltpu.sync_copy(w_hbm, w_v)

        def row_body(r, carry):
            def chunk_body(k, carry2):
                xvec = x_v[pl.ds(r * Lp + k * _LANES, _LANES)]
                xvec = jnp.where(xvec != xvec, jnp.float32(0.0), xvec)
                coff = k * _LANES * D
                for i in range(_LANES):
                    sv = jnp.full((_LANES,), xvec[i], jnp.float32)
                    for v in range(D // _LANES):
                        off = coff + i * D + v * _LANES
                        row_v[pl.ds(off, _LANES)] = sv * w_v[pl.ds(off, _LANES)]
                return carry2

            lax.fori_loop(0, n_chunks, chunk_body, 0)
            pltpu.sync_copy(
                row_v.at[pl.ds(0, LD)], out_hbm.at[pl.ds((base + r) * LD, LD)]
            )
            return carry

        lax.fori_loop(0, rows_per_w, row_body, 0)

    out_flat = sc_emb(xa.reshape(b * Lp), wflat)
    return out_flat.reshape(b, L + 1, D)


# speedup vs baseline: 1.1089x; 1.1089x over previous
import jax
import jax.numpy as jnp
from jax import lax
from jax.experimental import pallas as pl
from jax.experimental.pallas import tpu as pltpu
from jax.experimental.pallas import tpu_sc as plsc

_NC = 2
_NS = 16
_LANES = 16


def kernel(x, emb_weight, cls_token):
    b, L = x.shape
    D = emb_weight.shape[1]
    LD = (L + 1) * D
    nw = _NC * _NS
    rows_per_w = b // nw
    Lp = -(-(L + 1) // 8) * 8
    xa = jnp.concatenate(
        [x, jnp.ones((b, 1), jnp.float32), jnp.zeros((b, Lp - L - 1), jnp.float32)],
        axis=1,
    )
    wflat = jnp.concatenate(
        [
            emb_weight[1 : L + 1].reshape(L * D),
            cls_token.reshape(D),
            jnp.zeros((Lp - L - 1) * D, jnp.float32),
        ],
        axis=0,
    )
    LDp = Lp * D
    n_chunks = Lp // _LANES

    mesh = plsc.VectorSubcoreMesh(core_axis_name="c", subcore_axis_name="s")

    @pl.kernel(
        mesh=mesh,
        out_type=jax.ShapeDtypeStruct((b * LD,), jnp.float32),
        scratch_types=[
            pltpu.VMEM((rows_per_w * Lp,), jnp.float32),
            pltpu.VMEM((LDp,), jnp.float32),
            pltpu.VMEM((LDp,), jnp.float32),
            pltpu.VMEM((LDp,), jnp.float32),
            pltpu.SemaphoreType.DMA,
        ],
    )
    def sc_emb(xa_hbm, w_hbm, out_hbm, x_v, w_v, row_v0, row_v1, sem):
        wid = lax.axis_index("s") * _NC + lax.axis_index("c")
        base = wid * rows_per_w
        pltpu.sync_copy(xa_hbm.at[pl.ds(base * Lp, rows_per_w * Lp)], x_v)
        pltpu.sync_copy(w_hbm, w_v)

        def compute_row(r, buf):
            def chunk_body(k, carry2):
                xvec = x_v[pl.ds(r * Lp + k * _LANES, _LANES)]
                xvec = jnp.where(xvec != xvec, jnp.float32(0.0), xvec)
                coff = k * _LANES * D
                for i in range(_LANES):
                    sv = jnp.full((_LANES,), xvec[i], jnp.float32)
                    for v in range(D // _LANES):
                        off = coff + i * D + v * _LANES
                        buf[pl.ds(off, _LANES)] = sv * w_v[pl.ds(off, _LANES)]
                return carry2

            lax.fori_loop(0, n_chunks, chunk_body, 0)

        def start_out(r, buf):
            pltpu.make_async_copy(
                buf.at[pl.ds(0, LD)], out_hbm.at[pl.ds((base + r) * LD, LD)], sem
            ).start()

        def drain_one():
            pltpu.make_async_copy(
                row_v0.at[pl.ds(0, LD)], out_hbm.at[pl.ds(0, LD)], sem
            ).wait()

        def pair_body(p, carry):
            r0 = 2 * p
            compute_row(r0, row_v0)

            @pl.when(p >= 1)
            def _():
                drain_one()

            start_out(r0, row_v0)
            compute_row(r0 + 1, row_v1)
            drain_one()
            start_out(r0 + 1, row_v1)
            return carry

        lax.fori_loop(0, rows_per_w // 2, pair_body, 0)
        drain_one()

    out_flat = sc_emb(xa.reshape(b * Lp), wflat)
    return out_flat.reshape(b, L + 1, D)


# speedup vs baseline: 2.4918x; 2.2470x over previous
import jax
import jax.numpy as jnp
from jax.experimental import pallas as pl
from jax.experimental.pallas import tpu as pltpu

_ROW_BLOCK = 256
_N_CHUNKS = 4


def _chunk_bounds(Lp):
    step = -(-Lp // _N_CHUNKS)
    step += step % 2
    edges = list(range(0, Lp, step)) + [Lp]
    return list(zip(edges[:-1], edges[1:]))


def _emb_kernel(x_ref, *refs):
    m_refs, out_ref = refs[:-1], refs[-1]
    Lp = x_ref.shape[1]
    D = out_ref.shape[1] // Lp
    x = x_ref[...]
    xc = jnp.where(jnp.isnan(x), jnp.float32(0.0), x)
    for (l0, l1), m_ref in zip(_chunk_bounds(Lp), m_refs):
        out_ref[:, l0 * D : l1 * D] = jnp.dot(
            xc[:, l0:l1], m_ref[...], preferred_element_type=jnp.float32
        )


def kernel(x, emb_weight, cls_token):
    b, L = x.shape
    D = emb_weight.shape[1]
    xa = jnp.concatenate([x, jnp.ones((b, 1), jnp.float32)], axis=1)
    wrows = jnp.concatenate([emb_weight[1 : L + 1], cls_token.reshape(1, D)], axis=0)
    bounds = _chunk_bounds(L + 1)
    ms = []
    for l0, l1 in bounds:
        k = l1 - l0
        sel = (
            jnp.arange(k, dtype=jnp.int32)[:, None]
            == (jnp.arange(k * D, dtype=jnp.int32) // D)[None, :]
        )
        wc = wrows[l0:l1].reshape(1, k * D)
        ms.append(jnp.where(sel, jnp.tile(wc, (k, 1)), jnp.float32(0.0)))
    rb = _ROW_BLOCK
    m_specs = [
        pl.BlockSpec(m.shape, lambda i: (0, 0)) for m in ms
    ]
    out2d = pl.pallas_call(
        _emb_kernel,
        grid=(b // rb,),
        in_specs=[pl.BlockSpec((rb, L + 1), lambda i: (i, 0))] + m_specs,
        out_specs=pl.BlockSpec((rb, (L + 1) * D), lambda i: (i, 0)),
        out_shape=jax.ShapeDtypeStruct((b, (L + 1) * D), jnp.float32),
        compiler_params=pltpu.CompilerParams(
            dimension_semantics=("parallel",),
        ),
    )(xa, *ms)
    return out2d.reshape(b, L + 1, D)


# speedup vs baseline: 2.5196x; 1.0112x over previous
import jax
import jax.numpy as jnp
from jax.experimental import pallas as pl
from jax.experimental.pallas import tpu as pltpu

_ROW_BLOCK = 256
_N_CHUNKS = 4


def _chunk_bounds(L):
    step = -(-L // _N_CHUNKS)
    step += step % 2
    edges = list(range(0, L, step)) + [L]
    return list(zip(edges[:-1], edges[1:]))


def _emb_kernel(x_ref, cls_ref, *refs):
    m_refs, out_ref = refs[:-1], refs[-1]
    rb, L = x_ref.shape
    D = cls_ref.shape[1]
    x = x_ref[...]
    xc = jnp.where(jnp.isnan(x), jnp.float32(0.0), x)
    for (l0, l1), m_ref in zip(_chunk_bounds(L), m_refs):
        out_ref[:, l0 * D : l1 * D] = jnp.dot(
            xc[:, l0:l1], m_ref[...], preferred_element_type=jnp.float32
        )
    out_ref[:, L * D :] = jnp.broadcast_to(cls_ref[...], (rb, D))


def kernel(x, emb_weight, cls_token):
    b, L = x.shape
    D = emb_weight.shape[1]
    wrows = emb_weight[1 : L + 1]
    cls = cls_token.reshape(1, D)
    bounds = _chunk_bounds(L)
    ms = []
    for l0, l1 in bounds:
        k = l1 - l0
        sel = (
            jnp.arange(k, dtype=jnp.int32)[:, None]
            == (jnp.arange(k * D, dtype=jnp.int32) // D)[None, :]
        )
        wc = wrows[l0:l1].reshape(1, k * D)
        ms.append(jnp.where(sel, jnp.tile(wc, (k, 1)), jnp.float32(0.0)))
    rb = _ROW_BLOCK
    m_specs = [pl.BlockSpec(m.shape, lambda i: (0, 0)) for m in ms]
    out2d = pl.pallas_call(
        _emb_kernel,
        grid=(b // rb,),
        in_specs=[
            pl.BlockSpec((rb, L), lambda i: (i, 0)),
            pl.BlockSpec((1, D), lambda i: (0, 0)),
        ]
        + m_specs,
        out_specs=pl.BlockSpec((rb, (L + 1) * D), lambda i: (i, 0)),
        out_shape=jax.ShapeDtypeStruct((b, (L + 1) * D), jnp.float32),
        compiler_params=pltpu.CompilerParams(
            dimension_semantics=("parallel",),
        ),
    )(x, cls, *ms)
    return out2d.reshape(b, L + 1, D)
